# baseline (speedup 1.0000x reference)
import functools

import jax
import jax.numpy as jnp
from jax import lax
from jax.experimental import pallas as pl
from jax.experimental.pallas import tpu as pltpu
from jax.experimental.pallas import tpu_sc as plsc

VOCAB = 50257
HIDDEN = 768
MAX_LEN = 2048
BATCH = 4
SEQ = 2048

_INFO = plsc.get_sparse_core_info()
_NC = _INFO.num_cores
_NS = _INFO.num_subcores
_NW = _NC * _NS
_SPW = SEQ // _NW
_VECS = HIDDEN // 16
_CHUNK = 32
_NCHUNK = BATCH * _SPW // _CHUNK


def _emb_body(idx_hbm, word_hbm, pos_hbm, out_hbm,
              idx_v, rows0, rows1, pos_v, g0, g1, s0sem, s1sem):
    wid = lax.axis_index("s") * _NC + lax.axis_index("c")
    s0 = wid * _SPW

    pltpu.sync_copy(pos_hbm.at[pl.ds(s0, _SPW)], pos_v)
    for b in range(BATCH):
        pltpu.sync_copy(idx_hbm.at[pl.ds(b * SEQ + s0, _SPW)], idx_v.at[b])

    rows = (rows0, rows1)
    gsem = (g0, g1)
    ssem = (s0sem, s1sem)

    def chunk_info(c):
        b, h = divmod(c, _SPW // _CHUNK)
        out_row0 = b * SEQ + s0 + h * _CHUNK
        return b, h, out_row0

    def start_gather(c):
        b, h, _ = chunk_info(c)
        return pltpu.async_copy(
            word_hbm.at[idx_v.at[b, pl.ds(h * _CHUNK, _CHUNK)]],
            rows[c % 2], gsem[c % 2])

    gathers = {0: start_gather(0)}
    stores = {}
    for c in range(_NCHUNK):
        buf = c % 2
        if c + 1 < _NCHUNK:
            if c - 1 >= 0:
                stores[c - 1].wait()
            gathers[c + 1] = start_gather(c + 1)
        gathers[c].wait()

        b, h, out_row0 = chunk_info(c)
        p0 = h * _CHUNK

        def add_body(r, _, rv=rows[buf], p0=p0):
            for j in range(_VECS):
                col = j * 16
                rv[r, pl.ds(col, 16)] = (
                    rv[r, pl.ds(col, 16)] + pos_v[p0 + r, pl.ds(col, 16)]
                )
            return _

        lax.fori_loop(0, _CHUNK, add_body, 0)
        stores[c] = pltpu.async_copy(
            rows[buf], out_hbm.at[pl.ds(out_row0, _CHUNK)], ssem[buf])

    stores[_NCHUNK - 2].wait()
    stores[_NCHUNK - 1].wait()


@functools.partial(jax.jit, static_argnames=())
def _embed(idx_flat, word_table, pos_table):
    mesh = plsc.VectorSubcoreMesh(core_axis_name="c", subcore_axis_name="s")
    k = pl.kernel(
        _emb_body,
        out_type=jax.ShapeDtypeStruct((BATCH * SEQ, HIDDEN), jnp.float32),
        mesh=mesh,
        scratch_types=[
            pltpu.VMEM((BATCH, _SPW), jnp.int32),
            pltpu.VMEM((_CHUNK, HIDDEN), jnp.float32),
            pltpu.VMEM((_CHUNK, HIDDEN), jnp.float32),
            pltpu.VMEM((_SPW, HIDDEN), jnp.float32),
            pltpu.SemaphoreType.DMA,
            pltpu.SemaphoreType.DMA,
            pltpu.SemaphoreType.DMA,
            pltpu.SemaphoreType.DMA,
        ],
    )
    return k(idx_flat, word_table, pos_table)


def kernel(indices, word_table, pos_table):
    idx_flat = indices.reshape(-1)
    out = _embed(idx_flat, word_table, pos_table)
    return out.reshape(BATCH, SEQ, HIDDEN)


# speedup vs baseline: 1.2238x; 1.2238x over previous
import functools

import jax
import jax.numpy as jnp
from jax import lax
from jax.experimental import pallas as pl
from jax.experimental.pallas import tpu as pltpu
from jax.experimental.pallas import tpu_sc as plsc

VOCAB = 50257
HIDDEN = 768
MAX_LEN = 2048
BATCH = 4
SEQ = 2048

_INFO = plsc.get_sparse_core_info()
_NC = _INFO.num_cores
_NS = _INFO.num_subcores
_NW = _NC * _NS
_SPW = SEQ // _NW
_VECS = HIDDEN // 16


def _emb_body(idx_hbm, word_hbm, pos_hbm, out_hbm, idx_v, rows_v, pos_v, sem):
    wid = lax.axis_index("s") * _NC + lax.axis_index("c")
    s0 = wid * _SPW

    pltpu.sync_copy(pos_hbm.at[pl.ds(s0, _SPW)], pos_v)

    for b in range(BATCH):
        pltpu.sync_copy(idx_hbm.at[b, pl.ds(s0, _SPW)], idx_v)
        pltpu.async_copy(word_hbm.at[idx_v], rows_v, sem).wait()

        def add_body(r, _, rows_v=rows_v, pos_v=pos_v):
            for j in range(_VECS):
                c = j * 16
                rows_v[r, pl.ds(c, 16)] = (
                    rows_v[r, pl.ds(c, 16)] + pos_v[r, pl.ds(c, 16)]
                )
            return _

        lax.fori_loop(0, _SPW, add_body, 0)
        pltpu.sync_copy(rows_v, out_hbm.at[b, pl.ds(s0, _SPW)])


@functools.partial(jax.jit, static_argnames=())
def _embed(indices, word_table, pos_table):
    mesh = plsc.VectorSubcoreMesh(core_axis_name="c", subcore_axis_name="s")
    k = pl.kernel(
        _emb_body,
        out_type=jax.ShapeDtypeStruct((BATCH, SEQ, HIDDEN), jnp.float32),
        mesh=mesh,
        scratch_types=[
            pltpu.VMEM((_SPW,), jnp.int32),
            pltpu.VMEM((_SPW, HIDDEN), jnp.float32),
            pltpu.VMEM((_SPW, HIDDEN), jnp.float32),
            pltpu.SemaphoreType.DMA,
        ],
    )
    return k(indices, word_table, pos_table)


def kernel(indices, word_table, pos_table):
    return _embed(indices, word_table, pos_table)
